# baseline (speedup 1.0000x reference)
import functools

import jax
import jax.numpy as jnp
from jax import lax
from jax.experimental import pallas as pl
from jax.experimental.pallas import tpu as pltpu
from jax.experimental.pallas import tpu_sc as plsc

BATCH = 16384
D = 64
NC = 2
NS = 16
NW = NC * NS
BPW = BATCH // NW
L = 16
CH = 128
NCH = BPW // CH
MV_BR = 8192


def _mv_body(t_ref, w_ref, o_ref):
    o_ref[...] = jnp.dot(
        t_ref[...], w_ref[...], preferred_element_type=jnp.float32
    )


def _matvec(table, w):
    n = table.shape[0]
    grid = pl.cdiv(n, MV_BR)
    return pl.pallas_call(
        _mv_body,
        grid=(grid,),
        in_specs=[
            pl.BlockSpec((MV_BR, D), lambda i: (i, 0)),
            pl.BlockSpec((D,), lambda i: (0,)),
        ],
        out_specs=pl.BlockSpec((MV_BR,), lambda i: (i,)),
        out_shape=jax.ShapeDtypeStruct((n,), jnp.float32),
    )(table, w)


def _gather_body(uid_hbm, mid_hbm, su_hbm, sm_hbm, bb_hbm, out_hbm,
                 uidx, midx, sug, smg, bv, outv, usem, msem):
    wid = lax.axis_index("s") * NC + lax.axis_index("c")
    base = wid * BPW

    pltpu.sync_copy(uid_hbm.at[pl.ds(base, BPW)], uidx)
    pltpu.sync_copy(mid_hbm.at[pl.ds(base, BPW)], midx)
    pltpu.sync_copy(bb_hbm, bv)

    ucopies = [
        pltpu.async_copy(
            su_hbm.at[uidx.at[pl.ds(j * CH, CH)]],
            sug.at[pl.ds(j * CH, CH)], usem)
        for j in range(NCH)
    ]
    mcopies = [
        pltpu.async_copy(
            sm_hbm.at[midx.at[pl.ds(j * CH, CH)]],
            smg.at[pl.ds(j * CH, CH)], msem)
        for j in range(NCH)
    ]
    for c in ucopies:
        c.wait()
    for c in mcopies:
        c.wait()

    bvec = bv[...]
    for j in range(BPW // L):
        sl = pl.ds(j * L, L)
        outv[sl] = sug[sl] + smg[sl] + bvec

    pltpu.sync_copy(outv, out_hbm.at[pl.ds(base, BPW)])


@jax.jit
def _cf_call(user_ids, movie_ids, user_table, movie_table, wu, wm, bb):
    su = _matvec(user_table, wu)
    sm = _matvec(movie_table, wm)
    mesh = plsc.VectorSubcoreMesh(core_axis_name="c", subcore_axis_name="s")
    f = functools.partial(
        pl.kernel,
        mesh=mesh,
        compiler_params=pltpu.CompilerParams(
            needs_layout_passes=False, use_tc_tiling_on_sc=False
        ),
        out_type=jax.ShapeDtypeStruct((BATCH,), jnp.float32),
        scratch_types=[
            pltpu.VMEM((BPW,), jnp.int32),
            pltpu.VMEM((BPW,), jnp.int32),
            pltpu.VMEM((BPW,), jnp.float32),
            pltpu.VMEM((BPW,), jnp.float32),
            pltpu.VMEM((L,), jnp.float32),
            pltpu.VMEM((BPW,), jnp.float32),
            pltpu.SemaphoreType.DMA,
            pltpu.SemaphoreType.DMA,
        ],
    )(_gather_body)
    return f(user_ids, movie_ids, su, sm, bb)


def kernel(user_ids, movie_ids, user_table, movie_table, W, b):
    wu = W[:D, 0]
    wm = W[D:, 0]
    bb = jnp.broadcast_to(b.reshape(1), (L,))
    return _cf_call(
        user_ids.astype(jnp.int32), movie_ids.astype(jnp.int32),
        user_table, movie_table, wu, wm, bb,
    )


# speedup vs baseline: 1.4618x; 1.4618x over previous
import functools

import jax
import jax.numpy as jnp
from jax import lax
from jax.experimental import pallas as pl
from jax.experimental.pallas import tpu as pltpu
from jax.experimental.pallas import tpu_sc as plsc

BATCH = 16384
D = 64
NC = 2
NS = 16
NW = NC * NS
BPW = BATCH // NW
L = 16
CH = 128
NCH = BPW // CH
MV_BR = 8192


def _mv_body(t_ref, w_ref, o_ref):
    res = jax.lax.dot_general(
        w_ref[...], t_ref[...],
        (((1,), (1,)), ((), ())),
        preferred_element_type=jnp.float32,
    )
    o_ref[...] = res[0]


def _matvec(table, w8):
    n = table.shape[0]
    grid = pl.cdiv(n, MV_BR)
    return pl.pallas_call(
        _mv_body,
        grid=(grid,),
        in_specs=[
            pl.BlockSpec((MV_BR, D), lambda i: (i, 0)),
            pl.BlockSpec((8, D), lambda i: (0, 0)),
        ],
        out_specs=pl.BlockSpec((MV_BR,), lambda i: (i,)),
        out_shape=jax.ShapeDtypeStruct((n,), jnp.float32),
    )(table, w8)


def _gather_body(uid_hbm, mid_hbm, su_hbm, sm_hbm, bb_hbm, out_hbm,
                 uidx, midx, sug, smg, bv, outv, usem, msem):
    wid = lax.axis_index("s") * NC + lax.axis_index("c")
    base = wid * BPW

    pltpu.sync_copy(uid_hbm.at[pl.ds(base, BPW)], uidx)
    pltpu.sync_copy(mid_hbm.at[pl.ds(base, BPW)], midx)
    pltpu.sync_copy(bb_hbm, bv)

    ucopies = [
        pltpu.async_copy(
            su_hbm.at[uidx.at[pl.ds(j * CH, CH)]],
            sug.at[pl.ds(j * CH, CH)], usem)
        for j in range(NCH)
    ]
    mcopies = [
        pltpu.async_copy(
            sm_hbm.at[midx.at[pl.ds(j * CH, CH)]],
            smg.at[pl.ds(j * CH, CH)], msem)
        for j in range(NCH)
    ]
    for c in ucopies:
        c.wait()
    for c in mcopies:
        c.wait()

    bvec = bv[...]
    for j in range(BPW // L):
        sl = pl.ds(j * L, L)
        outv[sl] = sug[sl] + smg[sl] + bvec

    pltpu.sync_copy(outv, out_hbm.at[pl.ds(base, BPW)])


@jax.jit
def _cf_call(user_ids, movie_ids, user_table, movie_table, wu, wm, bb):
    su = _matvec(user_table, wu)
    sm = _matvec(movie_table, wm)
    mesh = plsc.VectorSubcoreMesh(core_axis_name="c", subcore_axis_name="s")
    f = functools.partial(
        pl.kernel,
        mesh=mesh,
        compiler_params=pltpu.CompilerParams(
            needs_layout_passes=False, use_tc_tiling_on_sc=False
        ),
        out_type=jax.ShapeDtypeStruct((BATCH,), jnp.float32),
        scratch_types=[
            pltpu.VMEM((BPW,), jnp.int32),
            pltpu.VMEM((BPW,), jnp.int32),
            pltpu.VMEM((BPW,), jnp.float32),
            pltpu.VMEM((BPW,), jnp.float32),
            pltpu.VMEM((L,), jnp.float32),
            pltpu.VMEM((BPW,), jnp.float32),
            pltpu.SemaphoreType.DMA,
            pltpu.SemaphoreType.DMA,
        ],
    )(_gather_body)
    return f(user_ids, movie_ids, su, sm, bb)


def kernel(user_ids, movie_ids, user_table, movie_table, W, b):
    wu = jnp.broadcast_to(W[:D, 0], (8, D))
    wm = jnp.broadcast_to(W[D:, 0], (8, D))
    bb = jnp.broadcast_to(b.reshape(1), (L,))
    return _cf_call(
        user_ids.astype(jnp.int32), movie_ids.astype(jnp.int32),
        user_table, movie_table, wu, wm, bb,
    )


# speedup vs baseline: 1.5389x; 1.0528x over previous
import functools

import jax
import jax.numpy as jnp
from jax import lax
from jax.experimental import pallas as pl
from jax.experimental.pallas import tpu as pltpu
from jax.experimental.pallas import tpu_sc as plsc

BATCH = 16384
D = 64
NC = 2
NS = 16
NW = NC * NS
BPW = BATCH // NW
L = 16
CH = 128
NCH = BPW // CH
MV_BR = 16384


def _mv_body(t_ref, w_ref, o_ref):
    res = jax.lax.dot_general(
        w_ref[...], t_ref[...],
        (((1,), (1,)), ((), ())),
        preferred_element_type=jnp.float32,
    )
    o_ref[...] = res[0]


def _matvec(table, w8):
    n = table.shape[0]
    grid = pl.cdiv(n, MV_BR)
    return pl.pallas_call(
        _mv_body,
        grid=(grid,),
        in_specs=[
            pl.BlockSpec((MV_BR, D), lambda i: (i, 0)),
            pl.BlockSpec((8, D), lambda i: (0, 0)),
        ],
        out_specs=pl.BlockSpec((MV_BR,), lambda i: (i,)),
        out_shape=jax.ShapeDtypeStruct((n,), jnp.float32),
    )(table, w8)


def _gather_body(uid_hbm, mid_hbm, su_hbm, sm_hbm, bb_hbm, out_hbm,
                 uidx, midx, sug, smg, bv, outv, usem, msem):
    wid = lax.axis_index("s") * NC + lax.axis_index("c")
    base = wid * BPW

    pltpu.sync_copy(uid_hbm.at[pl.ds(base, BPW)], uidx)
    pltpu.sync_copy(mid_hbm.at[pl.ds(base, BPW)], midx)
    pltpu.sync_copy(bb_hbm, bv)

    ucopies = [
        pltpu.async_copy(
            su_hbm.at[uidx.at[pl.ds(j * CH, CH)]],
            sug.at[pl.ds(j * CH, CH)], usem)
        for j in range(NCH)
    ]
    mcopies = [
        pltpu.async_copy(
            sm_hbm.at[midx.at[pl.ds(j * CH, CH)]],
            smg.at[pl.ds(j * CH, CH)], msem)
        for j in range(NCH)
    ]
    for c in ucopies:
        c.wait()
    for c in mcopies:
        c.wait()

    bvec = bv[...]
    for j in range(BPW // L):
        sl = pl.ds(j * L, L)
        outv[sl] = sug[sl] + smg[sl] + bvec

    pltpu.sync_copy(outv, out_hbm.at[pl.ds(base, BPW)])


@jax.jit
def _cf_call(user_ids, movie_ids, user_table, movie_table, wu, wm, bb):
    su = _matvec(user_table, wu)
    sm = _matvec(movie_table, wm)
    mesh = plsc.VectorSubcoreMesh(core_axis_name="c", subcore_axis_name="s")
    f = functools.partial(
        pl.kernel,
        mesh=mesh,
        compiler_params=pltpu.CompilerParams(
            needs_layout_passes=False, use_tc_tiling_on_sc=False
        ),
        out_type=jax.ShapeDtypeStruct((BATCH,), jnp.float32),
        scratch_types=[
            pltpu.VMEM((BPW,), jnp.int32),
            pltpu.VMEM((BPW,), jnp.int32),
            pltpu.VMEM((BPW,), jnp.float32),
            pltpu.VMEM((BPW,), jnp.float32),
            pltpu.VMEM((L,), jnp.float32),
            pltpu.VMEM((BPW,), jnp.float32),
            pltpu.SemaphoreType.DMA,
            pltpu.SemaphoreType.DMA,
        ],
    )(_gather_body)
    return f(user_ids, movie_ids, su, sm, bb)


def kernel(user_ids, movie_ids, user_table, movie_table, W, b):
    wu = jnp.broadcast_to(W[:D, 0], (8, D))
    wm = jnp.broadcast_to(W[D:, 0], (8, D))
    bb = jnp.broadcast_to(b.reshape(1), (L,))
    return _cf_call(
        user_ids.astype(jnp.int32), movie_ids.astype(jnp.int32),
        user_table, movie_table, wu, wm, bb,
    )


# speedup vs baseline: 1.9103x; 1.2413x over previous
import functools

import jax
import jax.numpy as jnp
from jax import lax
from jax.experimental import pallas as pl
from jax.experimental.pallas import tpu as pltpu
from jax.experimental.pallas import tpu_sc as plsc

BATCH = 16384
D = 64
NC = 2
NS = 16
NW = NC * NS
BPW = BATCH // NW
L = 16
BLK = 16
NBLK = BPW // BLK
NBUF = 2


def _cf_body(uid_hbm, mid_hbm, ut_hbm, mt_hbm, wb_hbm, out_hbm,
             uidx, midx, uring, mring, wv, outv, usem, msem):
    wid = lax.axis_index("s") * NC + lax.axis_index("c")
    base = wid * BPW

    pltpu.sync_copy(uid_hbm.at[pl.ds(base, BPW)], uidx)
    pltpu.sync_copy(mid_hbm.at[pl.ds(base, BPW)], midx)
    pltpu.sync_copy(wb_hbm, wv)

    lane = lax.iota(jnp.int32, L)
    last_lane = lane == (L - 1)

    def issue_block(blk, par):
        uids = uidx[pl.ds(blk * BLK, L)] & ~7
        mids = midx[pl.ds(blk * BLK, L)] & ~7
        for r in range(BLK):
            slot = par * BLK + r
            pltpu.async_copy(
                ut_hbm.at[pl.ds(pl.multiple_of(uids[r], 8), 8), :],
                uring.at[slot], usem)
            pltpu.async_copy(
                mt_hbm.at[pl.ds(pl.multiple_of(mids[r], 8), 8), :],
                mring.at[slot], msem)

    def wait_block(par):
        for r in range(BLK):
            slot = par * BLK + r
            pltpu.make_async_copy(
                ut_hbm.at[pl.ds(0, 8), :], uring.at[slot], usem).wait()
            pltpu.make_async_copy(
                mt_hbm.at[pl.ds(0, 8), :], mring.at[slot], msem).wait()

    issue_block(0, 0)

    w = [wv[pl.ds(k * L, L)] for k in range(2 * D // L)]
    bvec = wv[pl.ds(2 * D, L)]

    def block(g, carry):
        par = lax.rem(g, NBUF)
        wait_block(par)

        @pl.when(g + 1 < NBLK)
        def _():
            issue_block(g + 1, lax.rem(g + 1, NBUF))

        row0 = g * BLK
        usub = uidx[pl.ds(row0, L)] & 7
        msub = midx[pl.ds(row0, L)] & 7
        for r in range(BLK):
            slot = par * BLK + r
            ur = usub[r]
            mr = msub[r]
            acc = None
            for k in range(D // L):
                uv = uring[slot, ur, pl.ds(k * L, L)] * w[k]
                mv = mring[slot, mr, pl.ds(k * L, L)] * w[D // L + k]
                t = uv + mv
                acc = t if acc is None else acc + t
            s = plsc.cumsum(acc) + bvec
            plsc.store_scatter(
                outv, [jnp.full((L,), row0 + r, jnp.int32)], s, mask=last_lane
            )
        return carry

    lax.fori_loop(0, NBLK, block, 0)

    pltpu.sync_copy(outv, out_hbm.at[pl.ds(base, BPW)])


@jax.jit
def _cf_call(user_ids, movie_ids, user_table, movie_table, wb):
    mesh = plsc.VectorSubcoreMesh(core_axis_name="c", subcore_axis_name="s")
    f = functools.partial(
        pl.kernel,
        mesh=mesh,
        compiler_params=pltpu.CompilerParams(needs_layout_passes=False),
        out_type=jax.ShapeDtypeStruct((BATCH,), jnp.float32),
        scratch_types=[
            pltpu.VMEM((BPW,), jnp.int32),
            pltpu.VMEM((BPW,), jnp.int32),
            pltpu.VMEM((NBUF * BLK, 8, D), jnp.float32),
            pltpu.VMEM((NBUF * BLK, 8, D), jnp.float32),
            pltpu.VMEM((2 * D + L,), jnp.float32),
            pltpu.VMEM((BPW,), jnp.float32),
            pltpu.SemaphoreType.DMA,
            pltpu.SemaphoreType.DMA,
        ],
    )(_cf_body)
    return f(user_ids, movie_ids, user_table, movie_table, wb)


def kernel(user_ids, movie_ids, user_table, movie_table, W, b):
    wb = jnp.concatenate(
        [W.reshape(2 * D), jnp.broadcast_to(b.reshape(1), (L,))]
    )
    return _cf_call(
        user_ids.astype(jnp.int32), movie_ids.astype(jnp.int32),
        user_table, movie_table, wb,
    )
